# baseline (speedup 1.0000x reference)
import jax
import jax.numpy as jnp
from jax.experimental import pallas as pl

NUM_USERS = 30000
NUM_ITEMS = 70000
N = NUM_USERS + NUM_ITEMS
N_LAYERS = 2


def _l2norm(x):
    n = jnp.linalg.norm(x, axis=1, keepdims=True)
    return x / jnp.maximum(n, 1e-12)


def _add3_body(a_ref, b_ref, c_ref, o_ref):
    o_ref[...] = a_ref[...] + b_ref[...] + c_ref[...]


def _add3(a, b, c):
    return pl.pallas_call(
        _add3_body,
        out_shape=jax.ShapeDtypeStruct(a.shape, a.dtype),
        grid=(10,),
        in_specs=[pl.BlockSpec((N // 10, 32), lambda i: (i, 0))] * 3,
        out_specs=pl.BlockSpec((N // 10, 32), lambda i: (i, 0)),
    )(a, b, c)


def kernel(user_emb, item_emb, rows, cols):
    ones = jnp.ones((rows.shape[0],), dtype=jnp.float32)
    deg = jax.ops.segment_sum(ones, rows, num_segments=N)
    d_inv_sqrt = jnp.where(deg > 0, deg ** -0.5, 0.0)
    g_vals = d_inv_sqrt[rows] * d_inv_sqrt[cols]

    x0 = jnp.concatenate([user_emb, item_emb], axis=0)
    all_embs = [x0]
    user_fine = []
    item_fine = []
    for i in range(N_LAYERS):
        xi = all_embs[i]
        gnn = jax.ops.segment_sum(g_vals[:, None] * xi[cols], rows, num_segments=N)
        er = _l2norm(gnn[rows])
        ec = _l2norm(gnn[cols])
        scores = (jnp.sum(er * ec, axis=1) + 1.0) / 2.0
        rsum = jax.ops.segment_sum(scores, rows, num_segments=N)
        d_inv = jnp.where(rsum != 0, 1.0 / rsum, 0.0)
        fgv = d_inv[rows] * scores
        fine = jax.ops.segment_sum(fgv[:, None] * xi[cols], rows, num_segments=N)
        all_embs.append(gnn + fine)
        user_fine.append(fine[:NUM_USERS])
        item_fine.append(fine[NUM_USERS:])
    stacked = _add3(all_embs[0], all_embs[1], all_embs[2])
    u_embs = stacked[:NUM_USERS]
    i_embs = stacked[NUM_USERS:]
    return (u_embs, i_embs, *user_fine, *item_fine)


# speedup vs baseline: 5.0159x; 5.0159x over previous
import functools

import jax
import jax.numpy as jnp
from jax import lax
from jax.experimental import pallas as pl
from jax.experimental.pallas import tpu as pltpu
from jax.experimental.pallas import tpu_sc as plsc

NUM_USERS = 30000
NUM_ITEMS = 70000
N = NUM_USERS + NUM_ITEMS
E = 1600000
D = 32
H = 16
N_LAYERS = 2
NSC = 2
NT = 16
NW = NSC * NT
CH = 80

ETS = E // NW
NCHS = ETS // CH

RANGE = 40960
NRANGES = 3
ACCR = 41984
VST = ACCR // NT
VZR = 328
VNZ = VST // VZR

PR = 6400
PST = PR // NT
PZR = 80
PNZ = PST // PZR

RLENS = [RANGE, RANGE, N - 2 * RANGE]


def _zero_acc(zbuf, acc, s, zr, nz, stripe):
    for i in range(zr):
        zbuf[i, :] = jnp.zeros((H,), jnp.float32)
    for k in range(nz):
        pltpu.sync_copy(zbuf, acc.at[pl.ds(s * stripe + k * zr, zr)])


def _zero_acc_wide(zbuf, acc, s):
    for i in range(VZR):
        zbuf[i, pl.ds(0, H)] = jnp.zeros((H,), jnp.float32)
        zbuf[i, pl.ds(H, H)] = jnp.zeros((H,), jnp.float32)
    for k in range(VNZ):
        pltpu.sync_copy(zbuf, acc.at[pl.ds(s * VST + k * VZR, VZR)])


def _drain_acc(acc, out_hbm, c, s, rows_total, stripe):
    pltpu.sync_copy(
        acc.at[pl.ds(s * stripe, stripe)],
        out_hbm.at[pl.ds(c * rows_total + s * stripe, stripe)],
    )


def _remap_rows(rowsb, base, dump):
    for i in range(CH // H):
        v = rowsb[pl.ds(i * H, H)] - base
        inb = jnp.logical_and(v >= 0, v < RANGE)
        rowsb[pl.ds(i * H, H)] = jnp.where(inb, v, dump)


@functools.cache
def _build():
    mesh = plsc.VectorSubcoreMesh(
        core_axis_name="c", subcore_axis_name="s",
        num_cores=NSC, num_subcores=NT)
    cparams = pltpu.CompilerParams(use_tc_tiling_on_sc=False,
                                   needs_layout_passes=False)

    @functools.partial(
        pl.kernel,
        mesh=mesh,
        compiler_params=cparams,
        out_type=jax.ShapeDtypeStruct((NSC * PR, H), jnp.float32),
        scratch_types=[
            pltpu.VMEM((CH,), jnp.int32),
            pltpu.VMEM((CH,), jnp.int32),
            pltpu.VMEM((CH, H), jnp.float32),
            pltpu.VMEM((PZR, H), jnp.float32),
            pltpu.VMEM_SHARED((PR, H), jnp.float32),
            pltpu.SemaphoreType.DMA,
        ],
    )
    def deg_kernel(rows_hbm, out_hbm, rowsb, rowsdb, vbuf, zbuf, acc, sem):
        c = lax.axis_index("c")
        s = lax.axis_index("s")
        _zero_acc(zbuf, acc, s, PZR, PNZ, PST)
        plsc.subcore_barrier()
        lanes = lax.iota(jnp.int32, 16)
        base = (s * NSC + c) * ETS

        def body(k, carry):
            off = base + k * CH
            pltpu.sync_copy(rows_hbm.at[pl.ds(off, CH)], rowsb)
            for i in range(CH // H):
                v = rowsb[pl.ds(i * H, H)]
                rowsdb[pl.ds(i * H, H)] = lax.shift_right_logical(v, 4)
            for g in range(CH // H):
                rv16 = rowsb[pl.ds(g * H, H)]
                for l in range(H):
                    lane = lax.bitwise_and(rv16[l], 15)
                    vbuf[g * H + l, :] = jnp.where(lanes == lane, 1.0, 0.0)
            pltpu.sync_copy(vbuf, acc.at[rowsdb], add=True)
            return carry

        lax.fori_loop(0, NCHS, body, 0)
        plsc.subcore_barrier()
        _drain_acc(acc, out_hbm, c, s, PR, PST)

    def make_spmm(base_row):
        @functools.partial(
            pl.kernel,
            mesh=mesh,
            compiler_params=cparams,
            out_type=jax.ShapeDtypeStruct((NSC * ACCR, D), jnp.float32),
            scratch_types=[
                pltpu.VMEM((CH,), jnp.int32),
                pltpu.VMEM((CH,), jnp.int32),
                pltpu.VMEM((CH, D), jnp.float32),
                pltpu.VMEM((VZR, D), jnp.float32),
                pltpu.VMEM_SHARED((ACCR, D), jnp.float32),
                pltpu.SemaphoreType.DMA,
            ],
        )
        def spmm_kernel(rows_hbm, cols_hbm, tab_hbm, out_hbm,
                        rowsb, colsb, ybuf, zbuf, acc, sem):
            c = lax.axis_index("c")
            s = lax.axis_index("s")
            _zero_acc_wide(zbuf, acc, s)
            plsc.subcore_barrier()
            base = (s * NSC + c) * ETS
            dump = RANGE + s * 32 + c * 16

            def body(k, carry):
                off = base + k * CH
                pltpu.sync_copy(rows_hbm.at[pl.ds(off, CH)], rowsb)
                pltpu.sync_copy(cols_hbm.at[pl.ds(off, CH)], colsb)
                _remap_rows(rowsb, base_row, dump)
                pltpu.async_copy(tab_hbm.at[colsb], ybuf, sem).wait()
                pltpu.sync_copy(ybuf, acc.at[rowsb], add=True)
                return carry

            lax.fori_loop(0, NCHS, body, 0)
            plsc.subcore_barrier()
            _drain_acc(acc, out_hbm, c, s, ACCR, VST)

        return spmm_kernel

    @functools.partial(
        pl.kernel,
        mesh=mesh,
        compiler_params=cparams,
        out_type=[
            jax.ShapeDtypeStruct((E,), jnp.float32),
            jax.ShapeDtypeStruct((NSC * PR, H), jnp.float32),
        ],
        scratch_types=[
            pltpu.VMEM((CH,), jnp.int32),
            pltpu.VMEM((CH,), jnp.int32),
            pltpu.VMEM((CH,), jnp.int32),
            pltpu.VMEM((CH, D), jnp.float32),
            pltpu.VMEM((CH, D), jnp.float32),
            pltpu.VMEM((CH, H), jnp.float32),
            pltpu.VMEM((CH,), jnp.float32),
            pltpu.VMEM((PZR, H), jnp.float32),
            pltpu.VMEM_SHARED((PR, H), jnp.float32),
            pltpu.SemaphoreType.DMA,
        ],
    )
    def score_kernel(rows_hbm, cols_hbm, ghat_hbm, p_hbm, rsum_hbm,
                     rowsb, rowsdb, colsb, erbuf, ecbuf, pbuf, sbuf, zbuf,
                     acc, sem):
        c = lax.axis_index("c")
        s = lax.axis_index("s")
        _zero_acc(zbuf, acc, s, PZR, PNZ, PST)
        plsc.subcore_barrier()
        lanes = lax.iota(jnp.int32, 16)
        base = (s * NSC + c) * ETS

        def body(k, carry):
            off = base + k * CH
            pltpu.sync_copy(rows_hbm.at[pl.ds(off, CH)], rowsb)
            pltpu.sync_copy(cols_hbm.at[pl.ds(off, CH)], colsb)
            for i in range(CH // H):
                v = rowsb[pl.ds(i * H, H)]
                rowsdb[pl.ds(i * H, H)] = lax.shift_right_logical(v, 4)
            cp1 = pltpu.async_copy(ghat_hbm.at[rowsb], erbuf, sem)
            cp2 = pltpu.async_copy(ghat_hbm.at[colsb], ecbuf, sem)
            cp1.wait()
            cp2.wait()
            for g in range(CH // H):
                rv16 = rowsb[pl.ds(g * H, H)]
                pacc = jnp.zeros((H,), jnp.float32)
                for l in range(H):
                    e = g * H + l
                    prod = (erbuf[e, pl.ds(0, H)] * ecbuf[e, pl.ds(0, H)]
                            + erbuf[e, pl.ds(H, H)] * ecbuf[e, pl.ds(H, H)])
                    p = jnp.sum(prod)
                    pacc = jnp.where(lanes == l, p, pacc)
                    lane = lax.bitwise_and(rv16[l], 15)
                    pbuf[e, :] = jnp.where(lanes == lane, p, 0.0)
                sbuf[pl.ds(g * H, H)] = pacc
            pltpu.sync_copy(pbuf, acc.at[rowsdb], add=True)
            pltpu.sync_copy(sbuf, p_hbm.at[pl.ds(off, CH)])
            return carry

        lax.fori_loop(0, NCHS, body, 0)
        plsc.subcore_barrier()
        _drain_acc(acc, rsum_hbm, c, s, PR, PST)

    def make_fine(base_row):
        @functools.partial(
            pl.kernel,
            mesh=mesh,
            compiler_params=cparams,
            out_type=jax.ShapeDtypeStruct((NSC * ACCR, D), jnp.float32),
            scratch_types=[
                pltpu.VMEM((CH,), jnp.int32),
                pltpu.VMEM((CH,), jnp.int32),
                pltpu.VMEM((CH, D), jnp.float32),
                pltpu.VMEM((CH, D), jnp.float32),
                pltpu.VMEM((CH,), jnp.float32),
                pltpu.VMEM((VZR, D), jnp.float32),
                pltpu.VMEM_SHARED((ACCR, D), jnp.float32),
                pltpu.SemaphoreType.DMA,
            ],
        )
        def fine_kernel(rows_hbm, cols_hbm, tab_hbm, p_hbm, out_hbm,
                        rowsb, colsb, xbuf, vbuf, p0b, zbuf, acc, sem):
            c = lax.axis_index("c")
            s = lax.axis_index("s")
            _zero_acc_wide(zbuf, acc, s)
            plsc.subcore_barrier()
            base = (s * NSC + c) * ETS
            dump = RANGE + s * 32 + c * 16

            def body(k, carry):
                off = base + k * CH
                pltpu.sync_copy(rows_hbm.at[pl.ds(off, CH)], rowsb)
                pltpu.sync_copy(cols_hbm.at[pl.ds(off, CH)], colsb)
                _remap_rows(rowsb, base_row, dump)
                cp = pltpu.async_copy(tab_hbm.at[colsb], xbuf, sem)
                pltpu.sync_copy(p_hbm.at[pl.ds(off, CH)], p0b)
                for i in range(CH // H):
                    v = p0b[pl.ds(i * H, H)]
                    p0b[pl.ds(i * H, H)] = (v + 1.0) * 0.5
                cp.wait()
                for g in range(CH // H):
                    sv16 = p0b[pl.ds(g * H, H)]
                    for l in range(H):
                        e = g * H + l
                        vbuf[e, pl.ds(0, H)] = xbuf[e, pl.ds(0, H)] * sv16[l]
                        vbuf[e, pl.ds(H, H)] = xbuf[e, pl.ds(H, H)] * sv16[l]
                pltpu.sync_copy(vbuf, acc.at[rowsb], add=True)
                return carry

            lax.fori_loop(0, NCHS, body, 0)
            plsc.subcore_barrier()
            _drain_acc(acc, out_hbm, c, s, ACCR, VST)

        return fine_kernel

    spmms = tuple(make_spmm(r * RANGE) for r in range(NRANGES))
    fines = tuple(make_fine(r * RANGE) for r in range(NRANGES))
    return deg_kernel, spmms, score_kernel, fines


def _add3_body(a_ref, b_ref, c_ref, o_ref):
    o_ref[...] = a_ref[...] + b_ref[...] + c_ref[...]


def _add3(a, b, c):
    return pl.pallas_call(
        _add3_body,
        out_shape=jax.ShapeDtypeStruct(a.shape, a.dtype),
        grid=(10,),
        in_specs=[pl.BlockSpec((N // 10, D), lambda i: (i, 0))] * 3,
        out_specs=pl.BlockSpec((N // 10, D), lambda i: (i, 0)),
    )(a, b, c)


def _unpack_scalar(out):
    o = out.reshape(NSC, PR, H)
    return (o[0] + o[1]).reshape(PR * H)[:N]


def _merge_ranges(parts):
    segs = []
    for r, part in enumerate(parts):
        o = part.reshape(NSC, ACCR, D)
        segs.append((o[0] + o[1])[:RLENS[r]])
    return jnp.concatenate(segs, axis=0)


def kernel(user_emb, item_emb, rows, cols):
    deg_kernel, spmms, score_kernel, fines_k = _build()

    x0 = jnp.concatenate([user_emb, item_emb], axis=0)

    deg = _unpack_scalar(deg_kernel(rows))
    ds = jnp.where(deg > 0, lax.rsqrt(jnp.maximum(deg, 1e-30)), 0.0)
    dsc = ds[:, None]

    xi = x0
    outs_fine = []
    embs = [x0]
    for _ in range(N_LAYERS):
        ys = dsc * xi
        gnn = dsc * _merge_ranges([f(rows, cols, ys) for f in spmms])
        inv_nrm = 1.0 / jnp.maximum(
            jnp.sqrt(jnp.sum(gnn * gnn, axis=1)), 1e-12)
        ghat = inv_nrm[:, None] * gnn

        p_edge, rsum_raw = score_kernel(rows, cols, ghat)
        rsum = (_unpack_scalar(rsum_raw) + deg) * 0.5
        d_inv = jnp.where(rsum != 0, 1.0 / rsum, 0.0)

        fine = d_inv[:, None] * _merge_ranges(
            [f(rows, cols, xi, p_edge) for f in fines_k])

        xi = gnn + fine
        embs.append(xi)
        outs_fine.append(fine)

    stacked = _add3(embs[0], embs[1], embs[2])
    u_embs = stacked[:NUM_USERS]
    i_embs = stacked[NUM_USERS:]
    user_fine = [f[:NUM_USERS] for f in outs_fine]
    item_fine = [f[NUM_USERS:] for f in outs_fine]
    return (u_embs, i_embs, *user_fine, *item_fine)


# speedup vs baseline: 6.1841x; 1.2329x over previous
import functools

import jax
import jax.numpy as jnp
from jax import lax
from jax.experimental import pallas as pl
from jax.experimental.pallas import tpu as pltpu
from jax.experimental.pallas import tpu_sc as plsc

NUM_USERS = 30000
NUM_ITEMS = 70000
N = NUM_USERS + NUM_ITEMS
E = 1600000
D = 32
H = 16
N_LAYERS = 2
NSC = 2
NT = 16
NW = NSC * NT
CH = 80

ETS = E // NW
NCHS = ETS // CH

RANGE = 40960
NRANGES = 3
ACCR = 41984
VST = ACCR // NT
VZR = 328
VNZ = VST // VZR

PR = 6400
PST = PR // NT
PZR = 80
PNZ = PST // PZR

RLENS = [RANGE, RANGE, N - 2 * RANGE]


def _zero_acc(zbuf, acc, s, zr, nz, stripe):
    for i in range(zr):
        zbuf[i, :] = jnp.zeros((H,), jnp.float32)
    for k in range(nz):
        pltpu.sync_copy(zbuf, acc.at[pl.ds(s * stripe + k * zr, zr)])


def _zero_acc_wide(zbuf, acc, s):
    for i in range(VZR):
        zbuf[i, pl.ds(0, H)] = jnp.zeros((H,), jnp.float32)
        zbuf[i, pl.ds(H, H)] = jnp.zeros((H,), jnp.float32)
    for k in range(VNZ):
        pltpu.sync_copy(zbuf, acc.at[pl.ds(s * VST + k * VZR, VZR)])


def _drain_acc(acc, out_hbm, c, s, rows_total, stripe):
    pltpu.sync_copy(
        acc.at[pl.ds(s * stripe, stripe)],
        out_hbm.at[pl.ds(c * rows_total + s * stripe, stripe)],
    )


def _remap_rows(rowsb, base, dump):
    for i in range(CH // H):
        v = rowsb[pl.ds(i * H, H)] - base
        inb = jnp.logical_and(v >= 0, v < RANGE)
        rowsb[pl.ds(i * H, H)] = jnp.where(inb, v, dump)


@functools.cache
def _build():
    mesh = plsc.VectorSubcoreMesh(
        core_axis_name="c", subcore_axis_name="s",
        num_cores=NSC, num_subcores=NT)
    cparams = pltpu.CompilerParams(use_tc_tiling_on_sc=False,
                                   needs_layout_passes=False)

    @functools.partial(
        pl.kernel,
        mesh=mesh,
        compiler_params=cparams,
        out_type=jax.ShapeDtypeStruct((NSC * PR, H), jnp.float32),
        scratch_types=[
            pltpu.VMEM((CH,), jnp.int32),
            pltpu.VMEM((CH,), jnp.int32),
            pltpu.VMEM((CH, H), jnp.float32),
            pltpu.VMEM((PZR, H), jnp.float32),
            pltpu.VMEM_SHARED((PR, H), jnp.float32),
            pltpu.SemaphoreType.DMA,
        ],
    )
    def deg_kernel(rows_hbm, out_hbm, rowsb, rowsdb, vbuf, zbuf, acc, sem):
        c = lax.axis_index("c")
        s = lax.axis_index("s")
        _zero_acc(zbuf, acc, s, PZR, PNZ, PST)
        plsc.subcore_barrier()
        lanes = lax.iota(jnp.int32, 16)
        base = (s * NSC + c) * ETS

        def body(k, carry):
            off = base + k * CH
            pltpu.sync_copy(rows_hbm.at[pl.ds(off, CH)], rowsb)
            for i in range(CH // H):
                v = rowsb[pl.ds(i * H, H)]
                rowsdb[pl.ds(i * H, H)] = lax.shift_right_logical(v, 4)
            for g in range(CH // H):
                rv16 = rowsb[pl.ds(g * H, H)]
                for l in range(H):
                    lane = lax.bitwise_and(rv16[l], 15)
                    vbuf[g * H + l, :] = jnp.where(lanes == lane, 1.0, 0.0)
            pltpu.sync_copy(vbuf, acc.at[rowsdb], add=True)
            return carry

        lax.fori_loop(0, NCHS, body, 0)
        plsc.subcore_barrier()
        _drain_acc(acc, out_hbm, c, s, PR, PST)

    def make_spmm(base_row):
        @functools.partial(
            pl.kernel,
            mesh=mesh,
            compiler_params=cparams,
            out_type=jax.ShapeDtypeStruct((NSC * ACCR, D), jnp.float32),
            scratch_types=[
                pltpu.VMEM((CH,), jnp.int32),
                pltpu.VMEM((CH,), jnp.int32),
                pltpu.VMEM((CH, D), jnp.float32),
                pltpu.VMEM((CH,), jnp.int32),
                pltpu.VMEM((CH,), jnp.int32),
                pltpu.VMEM((CH, D), jnp.float32),
                pltpu.VMEM((VZR, D), jnp.float32),
                pltpu.VMEM_SHARED((ACCR, D), jnp.float32),
                pltpu.SemaphoreType.DMA,
                pltpu.SemaphoreType.DMA,
            ],
        )
        def spmm_kernel(rows_hbm, cols_hbm, tab_hbm, out_hbm,
                        rowsb0, colsb0, ybuf0, rowsb1, colsb1, ybuf1,
                        zbuf, acc, sem0, sem1):
            c = lax.axis_index("c")
            s = lax.axis_index("s")
            _zero_acc_wide(zbuf, acc, s)
            plsc.subcore_barrier()
            base = (s * NSC + c) * ETS
            dump = RANGE + s * 32 + c * 16

            def load(j, rowsb, colsb):
                jc = jnp.minimum(j, NCHS - 1)
                off = base + jc * CH
                pltpu.sync_copy(rows_hbm.at[pl.ds(off, CH)], rowsb)
                pltpu.sync_copy(cols_hbm.at[pl.ds(off, CH)], colsb)
                _remap_rows(rowsb, base_row, dump)

            load(0, rowsb0, colsb0)
            cp0 = pltpu.async_copy(tab_hbm.at[colsb0], ybuf0, sem0)

            def body(g, carry):
                a = 2 * g
                load(a + 1, rowsb1, colsb1)
                pltpu.async_copy(tab_hbm.at[colsb1], ybuf1, sem1)
                pltpu.make_async_copy(tab_hbm.at[colsb0], ybuf0, sem0).wait()
                pltpu.sync_copy(ybuf0, acc.at[rowsb0], add=True)
                load(a + 2, rowsb0, colsb0)
                pltpu.async_copy(tab_hbm.at[colsb0], ybuf0, sem0)
                pltpu.make_async_copy(tab_hbm.at[colsb1], ybuf1, sem1).wait()

                @pl.when(a + 1 < NCHS)
                def _():
                    pltpu.sync_copy(ybuf1, acc.at[rowsb1], add=True)

                return carry

            lax.fori_loop(0, (NCHS + 1) // 2, body, 0)
            pltpu.make_async_copy(tab_hbm.at[colsb0], ybuf0, sem0).wait()
            plsc.subcore_barrier()
            _drain_acc(acc, out_hbm, c, s, ACCR, VST)

        return spmm_kernel

    @functools.partial(
        pl.kernel,
        mesh=mesh,
        compiler_params=cparams,
        out_type=[
            jax.ShapeDtypeStruct((E,), jnp.float32),
            jax.ShapeDtypeStruct((NSC * PR, H), jnp.float32),
        ],
        scratch_types=[
            pltpu.VMEM((CH,), jnp.int32),
            pltpu.VMEM((CH,), jnp.int32),
            pltpu.VMEM((CH,), jnp.int32),
            pltpu.VMEM((CH, D), jnp.float32),
            pltpu.VMEM((CH, D), jnp.float32),
            pltpu.VMEM((CH, H), jnp.float32),
            pltpu.VMEM((CH,), jnp.float32),
            pltpu.VMEM((PZR, H), jnp.float32),
            pltpu.VMEM_SHARED((PR, H), jnp.float32),
            pltpu.SemaphoreType.DMA,
        ],
    )
    def score_kernel(rows_hbm, cols_hbm, ghat_hbm, p_hbm, rsum_hbm,
                     rowsb, rowsdb, colsb, erbuf, ecbuf, pbuf, sbuf, zbuf,
                     acc, sem):
        c = lax.axis_index("c")
        s = lax.axis_index("s")
        _zero_acc(zbuf, acc, s, PZR, PNZ, PST)
        plsc.subcore_barrier()
        lanes = lax.iota(jnp.int32, 16)
        base = (s * NSC + c) * ETS

        def body(k, carry):
            off = base + k * CH
            pltpu.sync_copy(rows_hbm.at[pl.ds(off, CH)], rowsb)
            pltpu.sync_copy(cols_hbm.at[pl.ds(off, CH)], colsb)
            for i in range(CH // H):
                v = rowsb[pl.ds(i * H, H)]
                rowsdb[pl.ds(i * H, H)] = lax.shift_right_logical(v, 4)
            cp1 = pltpu.async_copy(ghat_hbm.at[rowsb], erbuf, sem)
            cp2 = pltpu.async_copy(ghat_hbm.at[colsb], ecbuf, sem)
            cp1.wait()
            cp2.wait()
            for g in range(CH // H):
                rv16 = rowsb[pl.ds(g * H, H)]
                pacc = jnp.zeros((H,), jnp.float32)
                for l in range(H):
                    e = g * H + l
                    prod = (erbuf[e, pl.ds(0, H)] * ecbuf[e, pl.ds(0, H)]
                            + erbuf[e, pl.ds(H, H)] * ecbuf[e, pl.ds(H, H)])
                    p = jnp.sum(prod)
                    pacc = jnp.where(lanes == l, p, pacc)
                    lane = lax.bitwise_and(rv16[l], 15)
                    pbuf[e, :] = jnp.where(lanes == lane, p, 0.0)
                sbuf[pl.ds(g * H, H)] = pacc
            pltpu.sync_copy(pbuf, acc.at[rowsdb], add=True)
            pltpu.sync_copy(sbuf, p_hbm.at[pl.ds(off, CH)])
            return carry

        lax.fori_loop(0, NCHS, body, 0)
        plsc.subcore_barrier()
        _drain_acc(acc, rsum_hbm, c, s, PR, PST)

    def make_fine(base_row):
        @functools.partial(
            pl.kernel,
            mesh=mesh,
            compiler_params=cparams,
            out_type=jax.ShapeDtypeStruct((NSC * ACCR, D), jnp.float32),
            scratch_types=[
                pltpu.VMEM((CH,), jnp.int32),
                pltpu.VMEM((CH,), jnp.int32),
                pltpu.VMEM((CH, D), jnp.float32),
                pltpu.VMEM((CH,), jnp.float32),
                pltpu.VMEM((CH,), jnp.int32),
                pltpu.VMEM((CH,), jnp.int32),
                pltpu.VMEM((CH, D), jnp.float32),
                pltpu.VMEM((CH,), jnp.float32),
                pltpu.VMEM((CH, D), jnp.float32),
                pltpu.VMEM((VZR, D), jnp.float32),
                pltpu.VMEM_SHARED((ACCR, D), jnp.float32),
                pltpu.SemaphoreType.DMA,
                pltpu.SemaphoreType.DMA,
            ],
        )
        def fine_kernel(rows_hbm, cols_hbm, tab_hbm, p_hbm, out_hbm,
                        rowsb0, colsb0, xbuf0, p0b0, rowsb1, colsb1, xbuf1,
                        p0b1, vbuf, zbuf, acc, sem0, sem1):
            c = lax.axis_index("c")
            s = lax.axis_index("s")
            _zero_acc_wide(zbuf, acc, s)
            plsc.subcore_barrier()
            base = (s * NSC + c) * ETS
            dump = RANGE + s * 32 + c * 16

            def load(j, rowsb, colsb, p0b):
                jc = jnp.minimum(j, NCHS - 1)
                off = base + jc * CH
                pltpu.sync_copy(rows_hbm.at[pl.ds(off, CH)], rowsb)
                pltpu.sync_copy(cols_hbm.at[pl.ds(off, CH)], colsb)
                pltpu.sync_copy(p_hbm.at[pl.ds(off, CH)], p0b)
                _remap_rows(rowsb, base_row, dump)
                for i in range(CH // H):
                    v = p0b[pl.ds(i * H, H)]
                    p0b[pl.ds(i * H, H)] = (v + 1.0) * 0.5

            def mult_scatter(xbuf, p0b, rowsb):
                for g in range(CH // H):
                    sv16 = p0b[pl.ds(g * H, H)]
                    for l in range(H):
                        e = g * H + l
                        vbuf[e, pl.ds(0, H)] = xbuf[e, pl.ds(0, H)] * sv16[l]
                        vbuf[e, pl.ds(H, H)] = xbuf[e, pl.ds(H, H)] * sv16[l]
                pltpu.sync_copy(vbuf, acc.at[rowsb], add=True)

            load(0, rowsb0, colsb0, p0b0)
            pltpu.async_copy(tab_hbm.at[colsb0], xbuf0, sem0)

            def body(g, carry):
                a = 2 * g
                load(a + 1, rowsb1, colsb1, p0b1)
                pltpu.async_copy(tab_hbm.at[colsb1], xbuf1, sem1)
                pltpu.make_async_copy(tab_hbm.at[colsb0], xbuf0, sem0).wait()
                mult_scatter(xbuf0, p0b0, rowsb0)
                load(a + 2, rowsb0, colsb0, p0b0)
                pltpu.async_copy(tab_hbm.at[colsb0], xbuf0, sem0)
                pltpu.make_async_copy(tab_hbm.at[colsb1], xbuf1, sem1).wait()

                @pl.when(a + 1 < NCHS)
                def _():
                    mult_scatter(xbuf1, p0b1, rowsb1)

                return carry

            lax.fori_loop(0, (NCHS + 1) // 2, body, 0)
            pltpu.make_async_copy(tab_hbm.at[colsb0], xbuf0, sem0).wait()
            plsc.subcore_barrier()
            _drain_acc(acc, out_hbm, c, s, ACCR, VST)

        return fine_kernel

    spmms = tuple(make_spmm(r * RANGE) for r in range(NRANGES))
    fines = tuple(make_fine(r * RANGE) for r in range(NRANGES))
    return deg_kernel, spmms, score_kernel, fines


def _add3_body(a_ref, b_ref, c_ref, o_ref):
    o_ref[...] = a_ref[...] + b_ref[...] + c_ref[...]


def _add3(a, b, c):
    return pl.pallas_call(
        _add3_body,
        out_shape=jax.ShapeDtypeStruct(a.shape, a.dtype),
        grid=(10,),
        in_specs=[pl.BlockSpec((N // 10, D), lambda i: (i, 0))] * 3,
        out_specs=pl.BlockSpec((N // 10, D), lambda i: (i, 0)),
    )(a, b, c)


def _unpack_scalar(out):
    o = out.reshape(NSC, PR, H)
    return (o[0] + o[1]).reshape(PR * H)[:N]


def _merge_ranges(parts):
    segs = []
    for r, part in enumerate(parts):
        o = part.reshape(NSC, ACCR, D)
        segs.append((o[0] + o[1])[:RLENS[r]])
    return jnp.concatenate(segs, axis=0)


def kernel(user_emb, item_emb, rows, cols):
    deg_kernel, spmms, score_kernel, fines_k = _build()

    x0 = jnp.concatenate([user_emb, item_emb], axis=0)

    deg = _unpack_scalar(deg_kernel(rows))
    ds = jnp.where(deg > 0, lax.rsqrt(jnp.maximum(deg, 1e-30)), 0.0)
    dsc = ds[:, None]

    xi = x0
    outs_fine = []
    embs = [x0]
    for _ in range(N_LAYERS):
        ys = dsc * xi
        gnn = dsc * _merge_ranges([f(rows, cols, ys) for f in spmms])
        inv_nrm = 1.0 / jnp.maximum(
            jnp.sqrt(jnp.sum(gnn * gnn, axis=1)), 1e-12)
        ghat = inv_nrm[:, None] * gnn

        p_edge, rsum_raw = score_kernel(rows, cols, ghat)
        rsum = (_unpack_scalar(rsum_raw) + deg) * 0.5
        d_inv = jnp.where(rsum != 0, 1.0 / rsum, 0.0)

        fine = d_inv[:, None] * _merge_ranges(
            [f(rows, cols, xi, p_edge) for f in fines_k])

        xi = gnn + fine
        embs.append(xi)
        outs_fine.append(fine)

    stacked = _add3(embs[0], embs[1], embs[2])
    u_embs = stacked[:NUM_USERS]
    i_embs = stacked[NUM_USERS:]
    user_fine = [f[:NUM_USERS] for f in outs_fine]
    item_fine = [f[NUM_USERS:] for f in outs_fine]
    return (u_embs, i_embs, *user_fine, *item_fine)


# speedup vs baseline: 6.6230x; 1.0710x over previous
import functools

import jax
import jax.numpy as jnp
from jax import lax
from jax.experimental import pallas as pl
from jax.experimental.pallas import tpu as pltpu
from jax.experimental.pallas import tpu_sc as plsc

NUM_USERS = 30000
NUM_ITEMS = 70000
N = NUM_USERS + NUM_ITEMS
E = 1600000
D = 32
H = 16
N_LAYERS = 2
NSC = 2
NT = 16
NW = NSC * NT
CH = 80

ETS = E // NW
NCHS = ETS // CH

RANGE = 40960
NRANGES = 3
ACCR = 41984
VST = ACCR // NT
VZR = 328
VNZ = VST // VZR

PR = 6400
PST = PR // NT
PZR = 80
PNZ = PST // PZR

RLENS = [RANGE, RANGE, N - 2 * RANGE]


def _zero_acc(zbuf, acc, s, zr, nz, stripe):
    for i in range(zr):
        zbuf[i, :] = jnp.zeros((H,), jnp.float32)
    for k in range(nz):
        pltpu.sync_copy(zbuf, acc.at[pl.ds(s * stripe + k * zr, zr)])


def _zero_acc_wide(zbuf, acc, s):
    for i in range(VZR):
        zbuf[i, pl.ds(0, H)] = jnp.zeros((H,), jnp.float32)
        zbuf[i, pl.ds(H, H)] = jnp.zeros((H,), jnp.float32)
    for k in range(VNZ):
        pltpu.sync_copy(zbuf, acc.at[pl.ds(s * VST + k * VZR, VZR)])


def _drain_acc(acc, out_hbm, c, s, rows_total, stripe):
    pltpu.sync_copy(
        acc.at[pl.ds(s * stripe, stripe)],
        out_hbm.at[pl.ds(c * rows_total + s * stripe, stripe)],
    )


def _remap_rows(rowsb, base, dump):
    for i in range(CH // H):
        v = rowsb[pl.ds(i * H, H)] - base
        inb = jnp.logical_and(v >= 0, v < RANGE)
        rowsb[pl.ds(i * H, H)] = jnp.where(inb, v, dump)


@functools.cache
def _build():
    mesh = plsc.VectorSubcoreMesh(
        core_axis_name="c", subcore_axis_name="s",
        num_cores=NSC, num_subcores=NT)
    cparams = pltpu.CompilerParams(use_tc_tiling_on_sc=False,
                                   needs_layout_passes=False)

    @functools.partial(
        pl.kernel,
        mesh=mesh,
        compiler_params=cparams,
        out_type=jax.ShapeDtypeStruct((NSC * PR, H), jnp.float32),
        scratch_types=[
            pltpu.VMEM((CH,), jnp.int32),
            pltpu.VMEM((CH,), jnp.int32),
            pltpu.VMEM((CH, H), jnp.float32),
            pltpu.VMEM((PZR, H), jnp.float32),
            pltpu.VMEM_SHARED((PR, H), jnp.float32),
            pltpu.SemaphoreType.DMA,
        ],
    )
    def deg_kernel(rows_hbm, out_hbm, rowsb, rowsdb, vbuf, zbuf, acc, sem):
        c = lax.axis_index("c")
        s = lax.axis_index("s")
        _zero_acc(zbuf, acc, s, PZR, PNZ, PST)
        plsc.subcore_barrier()
        lanes = lax.iota(jnp.int32, 16)
        base = (s * NSC + c) * ETS

        def body(k, carry):
            off = base + k * CH
            pltpu.sync_copy(rows_hbm.at[pl.ds(off, CH)], rowsb)
            for i in range(CH // H):
                v = rowsb[pl.ds(i * H, H)]
                rowsdb[pl.ds(i * H, H)] = lax.shift_right_logical(v, 4)
            for g in range(CH // H):
                rv16 = rowsb[pl.ds(g * H, H)]
                for l in range(H):
                    lane = lax.bitwise_and(rv16[l], 15)
                    vbuf[g * H + l, :] = jnp.where(lanes == lane, 1.0, 0.0)
            pltpu.sync_copy(vbuf, acc.at[rowsdb], add=True)
            return carry

        lax.fori_loop(0, NCHS, body, 0)
        plsc.subcore_barrier()
        _drain_acc(acc, out_hbm, c, s, PR, PST)

    def make_spmm(base_row):
        @functools.partial(
            pl.kernel,
            mesh=mesh,
            compiler_params=cparams,
            out_type=jax.ShapeDtypeStruct((NSC * ACCR, D), jnp.float32),
            scratch_types=[
                pltpu.VMEM((CH,), jnp.int32),
                pltpu.VMEM((CH,), jnp.int32),
                pltpu.VMEM((CH, D), jnp.float32),
                pltpu.VMEM((CH,), jnp.int32),
                pltpu.VMEM((CH,), jnp.int32),
                pltpu.VMEM((CH, D), jnp.float32),
                pltpu.VMEM((VZR, D), jnp.float32),
                pltpu.VMEM_SHARED((ACCR, D), jnp.float32),
                pltpu.SemaphoreType.DMA,
                pltpu.SemaphoreType.DMA,
            ],
        )
        def spmm_kernel(rows_hbm, cols_hbm, tab_hbm, out_hbm,
                        rowsb0, colsb0, ybuf0, rowsb1, colsb1, ybuf1,
                        zbuf, acc, sem0, sem1):
            c = lax.axis_index("c")
            s = lax.axis_index("s")
            _zero_acc_wide(zbuf, acc, s)
            plsc.subcore_barrier()
            base = (s * NSC + c) * ETS
            dump = RANGE + s * 32 + c * 16

            def load(j, rowsb, colsb):
                jc = jnp.minimum(j, NCHS - 1)
                off = base + jc * CH
                pltpu.sync_copy(rows_hbm.at[pl.ds(off, CH)], rowsb)
                pltpu.sync_copy(cols_hbm.at[pl.ds(off, CH)], colsb)
                _remap_rows(rowsb, base_row, dump)

            load(0, rowsb0, colsb0)
            cp0 = pltpu.async_copy(tab_hbm.at[colsb0], ybuf0, sem0)

            def body(g, carry):
                a = 2 * g
                load(a + 1, rowsb1, colsb1)
                pltpu.async_copy(tab_hbm.at[colsb1], ybuf1, sem1)
                pltpu.make_async_copy(tab_hbm.at[colsb0], ybuf0, sem0).wait()
                pltpu.sync_copy(ybuf0, acc.at[rowsb0], add=True)
                load(a + 2, rowsb0, colsb0)
                pltpu.async_copy(tab_hbm.at[colsb0], ybuf0, sem0)
                pltpu.make_async_copy(tab_hbm.at[colsb1], ybuf1, sem1).wait()

                @pl.when(a + 1 < NCHS)
                def _():
                    pltpu.sync_copy(ybuf1, acc.at[rowsb1], add=True)

                return carry

            lax.fori_loop(0, (NCHS + 1) // 2, body, 0)
            pltpu.make_async_copy(tab_hbm.at[colsb0], ybuf0, sem0).wait()
            plsc.subcore_barrier()
            _drain_acc(acc, out_hbm, c, s, ACCR, VST)

        return spmm_kernel

    @functools.partial(
        pl.kernel,
        mesh=mesh,
        compiler_params=cparams,
        out_type=[
            jax.ShapeDtypeStruct((E,), jnp.float32),
            jax.ShapeDtypeStruct((NSC * PR, H), jnp.float32),
        ],
        scratch_types=[
            pltpu.VMEM((CH,), jnp.int32),
            pltpu.VMEM((CH,), jnp.int32),
            pltpu.VMEM((CH,), jnp.int32),
            pltpu.VMEM((CH, D), jnp.float32),
            pltpu.VMEM((CH, D), jnp.float32),
            pltpu.VMEM((CH,), jnp.int32),
            pltpu.VMEM((CH,), jnp.int32),
            pltpu.VMEM((CH,), jnp.int32),
            pltpu.VMEM((CH, D), jnp.float32),
            pltpu.VMEM((CH, D), jnp.float32),
            pltpu.VMEM((CH, H), jnp.float32),
            pltpu.VMEM((CH,), jnp.float32),
            pltpu.VMEM((PZR, H), jnp.float32),
            pltpu.VMEM_SHARED((PR, H), jnp.float32),
            pltpu.SemaphoreType.DMA,
            pltpu.SemaphoreType.DMA,
        ],
    )
    def score_kernel(rows_hbm, cols_hbm, ghat_hbm, p_hbm, rsum_hbm,
                     rowsb0, rowsdb0, colsb0, erbuf0, ecbuf0,
                     rowsb1, rowsdb1, colsb1, erbuf1, ecbuf1,
                     pbuf, sbuf, zbuf, acc, sem0, sem1):
        c = lax.axis_index("c")
        s = lax.axis_index("s")
        _zero_acc(zbuf, acc, s, PZR, PNZ, PST)
        plsc.subcore_barrier()
        lanes = lax.iota(jnp.int32, 16)
        base = (s * NSC + c) * ETS

        def load(j, rowsb, rowsdb, colsb, erbuf, ecbuf, sem):
            jc = jnp.minimum(j, NCHS - 1)
            off = base + jc * CH
            pltpu.sync_copy(rows_hbm.at[pl.ds(off, CH)], rowsb)
            pltpu.sync_copy(cols_hbm.at[pl.ds(off, CH)], colsb)
            for i in range(CH // H):
                v = rowsb[pl.ds(i * H, H)]
                rowsdb[pl.ds(i * H, H)] = lax.shift_right_logical(v, 4)
            pltpu.async_copy(ghat_hbm.at[rowsb], erbuf, sem)
            pltpu.async_copy(ghat_hbm.at[colsb], ecbuf, sem)

        def wait2(erbuf, ecbuf, sem):
            pltpu.make_async_copy(ghat_hbm.at[pl.ds(0, CH)], erbuf, sem).wait()
            pltpu.make_async_copy(ghat_hbm.at[pl.ds(0, CH)], ecbuf, sem).wait()

        def compute(j, rowsb, rowsdb, erbuf, ecbuf):
            jc = jnp.minimum(j, NCHS - 1)
            off = base + jc * CH
            for g in range(CH // H):
                rv16 = rowsb[pl.ds(g * H, H)]
                pacc = jnp.zeros((H,), jnp.float32)
                for l in range(H):
                    e = g * H + l
                    prod = (erbuf[e, pl.ds(0, H)] * ecbuf[e, pl.ds(0, H)]
                            + erbuf[e, pl.ds(H, H)] * ecbuf[e, pl.ds(H, H)])
                    p = jnp.sum(prod)
                    pacc = jnp.where(lanes == l, p, pacc)
                    lane = lax.bitwise_and(rv16[l], 15)
                    pbuf[e, :] = jnp.where(lanes == lane, p, 0.0)
                sbuf[pl.ds(g * H, H)] = pacc
            pltpu.sync_copy(pbuf, acc.at[rowsdb], add=True)
            pltpu.sync_copy(sbuf, p_hbm.at[pl.ds(off, CH)])

        load(0, rowsb0, rowsdb0, colsb0, erbuf0, ecbuf0, sem0)

        def body(g, carry):
            a = 2 * g
            load(a + 1, rowsb1, rowsdb1, colsb1, erbuf1, ecbuf1, sem1)
            wait2(erbuf0, ecbuf0, sem0)
            compute(a, rowsb0, rowsdb0, erbuf0, ecbuf0)
            load(a + 2, rowsb0, rowsdb0, colsb0, erbuf0, ecbuf0, sem0)
            wait2(erbuf1, ecbuf1, sem1)

            @pl.when(a + 1 < NCHS)
            def _():
                compute(a + 1, rowsb1, rowsdb1, erbuf1, ecbuf1)

            return carry

        lax.fori_loop(0, (NCHS + 1) // 2, body, 0)
        wait2(erbuf0, ecbuf0, sem0)
        plsc.subcore_barrier()
        _drain_acc(acc, rsum_hbm, c, s, PR, PST)

    def make_fine(base_row):
        @functools.partial(
            pl.kernel,
            mesh=mesh,
            compiler_params=cparams,
            out_type=jax.ShapeDtypeStruct((NSC * ACCR, D), jnp.float32),
            scratch_types=[
                pltpu.VMEM((CH,), jnp.int32),
                pltpu.VMEM((CH,), jnp.int32),
                pltpu.VMEM((CH, D), jnp.float32),
                pltpu.VMEM((CH,), jnp.float32),
                pltpu.VMEM((CH,), jnp.int32),
                pltpu.VMEM((CH,), jnp.int32),
                pltpu.VMEM((CH, D), jnp.float32),
                pltpu.VMEM((CH,), jnp.float32),
                pltpu.VMEM((CH, D), jnp.float32),
                pltpu.VMEM((VZR, D), jnp.float32),
                pltpu.VMEM_SHARED((ACCR, D), jnp.float32),
                pltpu.SemaphoreType.DMA,
                pltpu.SemaphoreType.DMA,
            ],
        )
        def fine_kernel(rows_hbm, cols_hbm, tab_hbm, p_hbm, out_hbm,
                        rowsb0, colsb0, xbuf0, p0b0, rowsb1, colsb1, xbuf1,
                        p0b1, vbuf, zbuf, acc, sem0, sem1):
            c = lax.axis_index("c")
            s = lax.axis_index("s")
            _zero_acc_wide(zbuf, acc, s)
            plsc.subcore_barrier()
            base = (s * NSC + c) * ETS
            dump = RANGE + s * 32 + c * 16

            def load(j, rowsb, colsb, p0b):
                jc = jnp.minimum(j, NCHS - 1)
                off = base + jc * CH
                pltpu.sync_copy(rows_hbm.at[pl.ds(off, CH)], rowsb)
                pltpu.sync_copy(cols_hbm.at[pl.ds(off, CH)], colsb)
                pltpu.sync_copy(p_hbm.at[pl.ds(off, CH)], p0b)
                _remap_rows(rowsb, base_row, dump)
                for i in range(CH // H):
                    v = p0b[pl.ds(i * H, H)]
                    p0b[pl.ds(i * H, H)] = (v + 1.0) * 0.5

            def mult_scatter(xbuf, p0b, rowsb):
                for g in range(CH // H):
                    sv16 = p0b[pl.ds(g * H, H)]
                    for l in range(H):
                        e = g * H + l
                        vbuf[e, pl.ds(0, H)] = xbuf[e, pl.ds(0, H)] * sv16[l]
                        vbuf[e, pl.ds(H, H)] = xbuf[e, pl.ds(H, H)] * sv16[l]
                pltpu.sync_copy(vbuf, acc.at[rowsb], add=True)

            load(0, rowsb0, colsb0, p0b0)
            pltpu.async_copy(tab_hbm.at[colsb0], xbuf0, sem0)

            def body(g, carry):
                a = 2 * g
                load(a + 1, rowsb1, colsb1, p0b1)
                pltpu.async_copy(tab_hbm.at[colsb1], xbuf1, sem1)
                pltpu.make_async_copy(tab_hbm.at[colsb0], xbuf0, sem0).wait()
                mult_scatter(xbuf0, p0b0, rowsb0)
                load(a + 2, rowsb0, colsb0, p0b0)
                pltpu.async_copy(tab_hbm.at[colsb0], xbuf0, sem0)
                pltpu.make_async_copy(tab_hbm.at[colsb1], xbuf1, sem1).wait()

                @pl.when(a + 1 < NCHS)
                def _():
                    mult_scatter(xbuf1, p0b1, rowsb1)

                return carry

            lax.fori_loop(0, (NCHS + 1) // 2, body, 0)
            pltpu.make_async_copy(tab_hbm.at[colsb0], xbuf0, sem0).wait()
            plsc.subcore_barrier()
            _drain_acc(acc, out_hbm, c, s, ACCR, VST)

        return fine_kernel

    spmms = tuple(make_spmm(r * RANGE) for r in range(NRANGES))
    fines = tuple(make_fine(r * RANGE) for r in range(NRANGES))
    return deg_kernel, spmms, score_kernel, fines


def _add3_body(a_ref, b_ref, c_ref, o_ref):
    o_ref[...] = a_ref[...] + b_ref[...] + c_ref[...]


def _add3(a, b, c):
    return pl.pallas_call(
        _add3_body,
        out_shape=jax.ShapeDtypeStruct(a.shape, a.dtype),
        grid=(10,),
        in_specs=[pl.BlockSpec((N // 10, D), lambda i: (i, 0))] * 3,
        out_specs=pl.BlockSpec((N // 10, D), lambda i: (i, 0)),
    )(a, b, c)


def _unpack_scalar(out):
    o = out.reshape(NSC, PR, H)
    return (o[0] + o[1]).reshape(PR * H)[:N]


def _merge_ranges(parts):
    segs = []
    for r, part in enumerate(parts):
        o = part.reshape(NSC, ACCR, D)
        segs.append((o[0] + o[1])[:RLENS[r]])
    return jnp.concatenate(segs, axis=0)


def kernel(user_emb, item_emb, rows, cols):
    deg_kernel, spmms, score_kernel, fines_k = _build()

    x0 = jnp.concatenate([user_emb, item_emb], axis=0)

    deg = _unpack_scalar(deg_kernel(rows))
    ds = jnp.where(deg > 0, lax.rsqrt(jnp.maximum(deg, 1e-30)), 0.0)
    dsc = ds[:, None]

    xi = x0
    outs_fine = []
    embs = [x0]
    for _ in range(N_LAYERS):
        ys = dsc * xi
        gnn = dsc * _merge_ranges([f(rows, cols, ys) for f in spmms])
        inv_nrm = 1.0 / jnp.maximum(
            jnp.sqrt(jnp.sum(gnn * gnn, axis=1)), 1e-12)
        ghat = inv_nrm[:, None] * gnn

        p_edge, rsum_raw = score_kernel(rows, cols, ghat)
        rsum = (_unpack_scalar(rsum_raw) + deg) * 0.5
        d_inv = jnp.where(rsum != 0, 1.0 / rsum, 0.0)

        fine = d_inv[:, None] * _merge_ranges(
            [f(rows, cols, xi, p_edge) for f in fines_k])

        xi = gnn + fine
        embs.append(xi)
        outs_fine.append(fine)

    stacked = _add3(embs[0], embs[1], embs[2])
    u_embs = stacked[:NUM_USERS]
    i_embs = stacked[NUM_USERS:]
    user_fine = [f[:NUM_USERS] for f in outs_fine]
    item_fine = [f[NUM_USERS:] for f in outs_fine]
    return (u_embs, i_embs, *user_fine, *item_fine)
